# baseline (speedup 1.0000x reference)
import jax
import jax.numpy as jnp
from jax import lax
from jax.experimental import pallas as pl
from jax.experimental.pallas import tpu as pltpu
from jax.experimental.pallas import tpu_sc as plsc

N = 10000
E = 320000
G = 64
H = 128

NC = 2
NS = 16
NW = NC * NS
L = 16

DP = 144
EW = E // NW
B = 80
BH = B // 2
NCHUNK = EW // B
RPT = N // NS

_f32 = jnp.float32


def _edge_body(xl_h, xr_h, src_h, dst_h, att_h, zer_h, acc_h,
               att_v, idx_s0, idx_s1, idx_d0, idx_d1, sidx0, sidx1,
               wrr0, wrr1, xra, xrb, exb, acc_sh,
               sem_ix0, sem_ix1, sem_xl0, sem_xl1, sem_xra, sem_xrb,
               sem_sc0, sem_sc1):
  c = lax.axis_index("c")
  s = lax.axis_index("s")
  wid = c * NS + s

  idx_s = [idx_s0, idx_s1]
  idx_d = [idx_d0, idx_d1]
  sidx = [sidx0, sidx1]
  wrr = [wrr0, wrr1]
  sem_ix = [sem_ix0, sem_ix1]
  sem_xl = [sem_xl0, sem_xl1]
  sem_sc = [sem_sc0, sem_sc1]

  pltpu.sync_copy(zer_h.at[pl.ds(s * RPT, RPT)],
                  acc_sh.at[pl.ds(s * RPT, RPT)])
  pltpu.sync_copy(att_h, att_v)
  plsc.subcore_barrier()

  att_c = [att_v[pl.ds(k * L, L)] for k in range(8)]

  def issue_idx(cj, b):
    pltpu.async_copy(src_h.at[wid, cj], idx_s[b], sem_ix[b])
    pltpu.async_copy(dst_h.at[wid, cj], idx_d[b], sem_ix[b])

  def wait_idx(b):
    pltpu.make_async_copy(src_h.at[wid, 0], idx_s[b], sem_ix[b]).wait()
    pltpu.make_async_copy(dst_h.at[wid, 0], idx_d[b], sem_ix[b]).wait()

  def issue_xl(b):
    pltpu.async_copy(xl_h.at[idx_s[b]], wrr[b], sem_xl[b])

  def wait_xl(b):
    pltpu.make_async_copy(xl_h.at[idx_s[b]], wrr[b], sem_xl[b]).wait()

  def issue_xr(b, h, xr_v, sem):
    pltpu.async_copy(xr_h.at[idx_d[b].at[pl.ds(h * BH, BH)]], xr_v, sem)

  def wait_xr(b, h, xr_v, sem):
    pltpu.make_async_copy(xr_h.at[idx_d[b].at[pl.ds(h * BH, BH)]], xr_v,
                          sem).wait()

  def issue_scatter(b, h):
    pltpu.async_copy(wrr[b].at[pl.ds(h * BH, BH)], acc_sh.at[sidx[b].at[h]],
                     sem_sc[b], add=True)

  def wait_scatter(b, h):
    pltpu.make_async_copy(wrr[b].at[pl.ds(h * BH, BH)],
                          acc_sh.at[sidx[b].at[h]], sem_sc[b]).wait()

  def copy_sidx(b):
    for h in range(2):
      for o in (0, L, BH - L):
        sidx[b][h, pl.ds(o, L)] = idx_d[b][pl.ds(h * BH + o, L)]

  def compute_half(b, h, xr_v):
    w = wrr[b]

    def score_body(i, carry):
      xi = i - h * BH
      p = []
      for k in range(8):
        sv = w[i, pl.ds(k * L, L)] + xr_v[xi, pl.ds(k * L, L)]
        p.append(att_c[k] * jnp.maximum(sv, 0.2 * sv))
      acc = ((p[0] + p[1]) + (p[2] + p[3])) + ((p[4] + p[5]) + (p[6] + p[7]))
      exb[i, pl.ds(0, L)] = jnp.full((L,), jnp.sum(acc), _f32)
      return carry

    def scale_body(i, carry):
      ex = jnp.exp(exb[i, pl.ds(0, L)])
      for k in range(9):
        w[i, pl.ds(k * L, L)] = w[i, pl.ds(k * L, L)] * ex
      return carry

    lax.fori_loop(h * BH, h * BH + BH, score_body, 0, unroll=2)
    lax.fori_loop(h * BH, h * BH + BH, scale_body, 0, unroll=4)

  issue_idx(0, 0)
  issue_idx(1, 1)
  wait_idx(0)
  issue_xl(0)
  issue_xr(0, 0, xra, sem_xra)
  issue_xr(0, 1, xrb, sem_xrb)

  def step(cj, b, tail):
    nb = 1 - b
    if not tail:
      wait_idx(nb)

    @pl.when(cj >= 1)
    def _():
      wait_scatter(nb, 0)
      wait_scatter(nb, 1)

    if not tail:
      issue_xl(nb)
    wait_xl(b)
    copy_sidx(b)
    wait_xr(b, 0, xra, sem_xra)
    compute_half(b, 0, xra)
    issue_scatter(b, 0)
    if not tail:
      issue_xr(nb, 0, xra, sem_xra)
    wait_xr(b, 1, xrb, sem_xrb)
    compute_half(b, 1, xrb)
    issue_scatter(b, 1)
    if not tail:
      issue_xr(nb, 1, xrb, sem_xrb)

      @pl.when(cj <= NCHUNK - 3)
      def _():
        issue_idx(cj + 2, b)

  @pl.loop(0, NCHUNK - 1, step=2)
  def pair_body(j):
    step(j, 0, False)
    step(j + 1, 1, False)

  step(NCHUNK - 1, (NCHUNK - 1) % 2, True)
  tb = (NCHUNK - 1) % 2
  wait_scatter(tb, 0)
  wait_scatter(tb, 1)

  plsc.subcore_barrier()
  pltpu.sync_copy(acc_sh.at[pl.ds(s * RPT, RPT)],
                  acc_h.at[c, pl.ds(s * RPT, RPT)])


@jax.jit
def _edge_phase(xl, xr, src3, dst3, att, zeros):
  mesh = plsc.VectorSubcoreMesh(core_axis_name="c", subcore_axis_name="s",
                                num_cores=NC, num_subcores=NS)
  kern = pl.kernel(
      _edge_body,
      out_type=jax.ShapeDtypeStruct((NC, N, DP), _f32),
      mesh=mesh,
      scratch_types=(
          [pltpu.VMEM((H,), _f32)]
          + [pltpu.VMEM((B,), jnp.int32)] * 4
          + [pltpu.VMEM((2, BH), jnp.int32)] * 2
          + [pltpu.VMEM((B, DP), _f32)] * 2
          + [pltpu.VMEM((BH, H), _f32)] * 2
          + [pltpu.VMEM((B, L), _f32)]
          + [pltpu.VMEM_SHARED((N, DP), _f32)]
          + [pltpu.SemaphoreType.DMA] * 8
      ),
      compiler_params=pltpu.CompilerParams(use_tc_tiling_on_sc=False,
                                           needs_layout_passes=False),
  )
  return kern(xl, xr, src3, dst3, att, zeros)


def _pad_cols(xl):
  marker = (lax.broadcasted_iota(jnp.int32, (N, DP - H), 1) == 0
            ).astype(_f32)
  return jnp.concatenate([xl, marker], axis=1)


def _proj_body(x_ref, wl_ref, wr_ref, xl_ref, xr_ref):
  x = x_ref[...]
  xl_ref[...] = _pad_cols(jnp.dot(x, wl_ref[...], preferred_element_type=_f32))
  xr_ref[...] = jnp.dot(x, wr_ref[...], preferred_element_type=_f32)


def _proj(x, wl, wr):
  return pl.pallas_call(
      _proj_body,
      out_shape=(jax.ShapeDtypeStruct((N, DP), _f32),
                 jax.ShapeDtypeStruct((N, H), _f32)),
  )(x, wl, wr)


def _combine_proj_body(acc_ref, b_ref, wl_ref, wr_ref, xl_ref, xr_ref):
  num = acc_ref[0, :, :H] + acc_ref[1, :, :H]
  den = acc_ref[0, :, H:H + 1] + acc_ref[1, :, H:H + 1]
  h = jnp.where(den > 0.0, num / den, 0.0) + b_ref[...]
  h = jnp.maximum(h, 0.0)
  xl_ref[...] = _pad_cols(jnp.dot(h, wl_ref[...], preferred_element_type=_f32))
  xr_ref[...] = jnp.dot(h, wr_ref[...], preferred_element_type=_f32)


def _combine_proj(acc, b, wl, wr):
  return pl.pallas_call(
      _combine_proj_body,
      out_shape=(jax.ShapeDtypeStruct((N, DP), _f32),
                 jax.ShapeDtypeStruct((N, H), _f32)),
  )(acc, b, wl, wr)


def _head_body(acc_ref, b_ref, batch_ref, wm1_ref, bm1_ref, wm2_ref,
               bm2_ref, wm3_ref, bm3_ref, out_ref):
  num = acc_ref[0, :, :H] + acc_ref[1, :, :H]
  den = acc_ref[0, :, H:H + 1] + acc_ref[1, :, H:H + 1]
  h = jnp.where(den > 0.0, num / den, 0.0) + b_ref[...]
  bi = batch_ref[...]
  seg = lax.broadcasted_iota(jnp.int32, (G, N), 0)
  mask = (bi == seg).astype(_f32)
  sums = jnp.dot(mask, h, preferred_element_type=_f32)
  cnt = jnp.sum(mask, axis=1, keepdims=True)
  pooled = sums / jnp.maximum(cnt, 1.0)
  z = jax.nn.sigmoid(jnp.dot(pooled, wm1_ref[...],
                             preferred_element_type=_f32) + bm1_ref[...])
  z = jax.nn.sigmoid(jnp.dot(z, wm2_ref[...],
                             preferred_element_type=_f32) + bm2_ref[...])
  out_ref[...] = jnp.dot(z, wm3_ref[...],
                         preferred_element_type=_f32) + bm3_ref[...]


def _head(acc, b3, batch, wm1, bm1, wm2, bm2, wm3, bm3):
  return pl.pallas_call(
      _head_body,
      out_shape=jax.ShapeDtypeStruct((G, 2), _f32),
  )(acc, b3, batch, wm1, bm1, wm2, bm2, wm3, bm3)


def kernel(x, edge_index, batch, Wl1, Wr1, att1, b1, Wl2, Wr2, att2, b2,
           Wl3, Wr3, att3, b3, Wm1, bm1, Wm2, bm2, Wm3, bm3):
  src3 = edge_index[0].reshape(NW, NCHUNK, B)
  dst3 = edge_index[1].reshape(NW, NCHUNK, B)
  zeros = jnp.zeros((N, DP), _f32)

  xl, xr = _proj(x, Wl1, Wr1)
  acc = _edge_phase(xl, xr, src3, dst3, att1, zeros)
  xl, xr = _combine_proj(acc, b1.reshape(1, H), Wl2, Wr2)
  acc = _edge_phase(xl, xr, src3, dst3, att2, zeros)
  xl, xr = _combine_proj(acc, b2.reshape(1, H), Wl3, Wr3)
  acc = _edge_phase(xl, xr, src3, dst3, att3, zeros)
  return _head(acc, b3.reshape(1, H), batch.reshape(1, N),
               Wm1, bm1.reshape(1, -1), Wm2, bm2.reshape(1, -1),
               Wm3, bm3.reshape(1, 2))


# speedup vs baseline: 1.1573x; 1.1573x over previous
import jax
import jax.numpy as jnp
from jax import lax
from jax.experimental import pallas as pl
from jax.experimental.pallas import tpu as pltpu
from jax.experimental.pallas import tpu_sc as plsc

N = 10000
E = 320000
G = 64
H = 128

NC = 2
NS = 16
NW = NC * NS
L = 16

DP = 144
EW = E // NW
B = 80
BH = B // 2
NCHUNK = EW // B
RPT = N // NS

_f32 = jnp.float32


def _edge_body(xl_h, xr_h, src_h, dst_h, att_h, zer_h, acc_h,
               att_v, idx_s0, idx_s1, idx_d0, idx_d1, sidx0, sidx1,
               wrr0, wrr1, xra, xrb, acc_sh,
               sem_ix0, sem_ix1, sem_xl0, sem_xl1, sem_xra, sem_xrb,
               sem_sc0, sem_sc1):
  c = lax.axis_index("c")
  s = lax.axis_index("s")
  wid = c * NS + s

  idx_s = [idx_s0, idx_s1]
  idx_d = [idx_d0, idx_d1]
  sidx = [sidx0, sidx1]
  wrr = [wrr0, wrr1]
  sem_ix = [sem_ix0, sem_ix1]
  sem_xl = [sem_xl0, sem_xl1]
  sem_sc = [sem_sc0, sem_sc1]

  pltpu.sync_copy(zer_h.at[pl.ds(s * RPT, RPT)],
                  acc_sh.at[pl.ds(s * RPT, RPT)])
  pltpu.sync_copy(att_h, att_v)
  plsc.subcore_barrier()

  att_c = [att_v[pl.ds(k * L, L)] for k in range(8)]

  def issue_idx(cj, b):
    pltpu.async_copy(src_h.at[wid, cj], idx_s[b], sem_ix[b])
    pltpu.async_copy(dst_h.at[wid, cj], idx_d[b], sem_ix[b])

  def wait_idx(b):
    pltpu.make_async_copy(src_h.at[wid, 0], idx_s[b], sem_ix[b]).wait()
    pltpu.make_async_copy(dst_h.at[wid, 0], idx_d[b], sem_ix[b]).wait()

  def issue_xl(b):
    pltpu.async_copy(xl_h.at[idx_s[b]], wrr[b], sem_xl[b])

  def wait_xl(b):
    pltpu.make_async_copy(xl_h.at[idx_s[b]], wrr[b], sem_xl[b]).wait()

  def issue_xr(b, h, xr_v, sem):
    pltpu.async_copy(xr_h.at[idx_d[b].at[pl.ds(h * BH, BH)]], xr_v, sem)

  def wait_xr(b, h, xr_v, sem):
    pltpu.make_async_copy(xr_h.at[idx_d[b].at[pl.ds(h * BH, BH)]], xr_v,
                          sem).wait()

  def issue_scatter(b, h):
    pltpu.async_copy(wrr[b].at[pl.ds(h * BH, BH)], acc_sh.at[sidx[b].at[h]],
                     sem_sc[b], add=True)

  def wait_scatter(b, h):
    pltpu.make_async_copy(wrr[b].at[pl.ds(h * BH, BH)],
                          acc_sh.at[sidx[b].at[h]], sem_sc[b]).wait()

  def copy_sidx(b):
    for h in range(2):
      for o in (0, L, BH - L):
        sidx[b][h, pl.ds(o, L)] = idx_d[b][pl.ds(h * BH + o, L)]

  def compute_half(b, h, xr_v):
    w = wrr[b]

    def edge_body(i, carry):
      vl = [w[i, pl.ds(k * L, L)] for k in range(8)]
      xi = i - h * BH
      p = []
      for k in range(8):
        sv = vl[k] + xr_v[xi, pl.ds(k * L, L)]
        p.append(att_c[k] * jnp.maximum(sv, 0.2 * sv))
      acc = ((p[0] + p[1]) + (p[2] + p[3])) + ((p[4] + p[5]) + (p[6] + p[7]))
      ex = jnp.exp(jnp.full((L,), jnp.sum(acc), _f32))
      for k in range(8):
        w[i, pl.ds(k * L, L)] = vl[k] * ex
      w[i, pl.ds(8 * L, L)] = ex
      return carry

    lax.fori_loop(h * BH, h * BH + BH, edge_body, 0, unroll=2)

  issue_idx(0, 0)
  issue_idx(1, 1)
  wait_idx(0)
  issue_xl(0)
  issue_xr(0, 0, xra, sem_xra)
  issue_xr(0, 1, xrb, sem_xrb)

  def step(cj, b, tail):
    nb = 1 - b
    if not tail:
      wait_idx(nb)

    @pl.when(cj >= 1)
    def _():
      wait_scatter(nb, 0)
      wait_scatter(nb, 1)

    if not tail:
      issue_xl(nb)
    wait_xl(b)
    copy_sidx(b)
    wait_xr(b, 0, xra, sem_xra)
    compute_half(b, 0, xra)
    issue_scatter(b, 0)
    if not tail:
      issue_xr(nb, 0, xra, sem_xra)
    wait_xr(b, 1, xrb, sem_xrb)
    compute_half(b, 1, xrb)
    issue_scatter(b, 1)
    if not tail:
      issue_xr(nb, 1, xrb, sem_xrb)

      @pl.when(cj <= NCHUNK - 3)
      def _():
        issue_idx(cj + 2, b)

  @pl.loop(0, NCHUNK - 1, step=2)
  def pair_body(j):
    step(j, 0, False)
    step(j + 1, 1, False)

  step(NCHUNK - 1, (NCHUNK - 1) % 2, True)
  tb = (NCHUNK - 1) % 2
  wait_scatter(tb, 0)
  wait_scatter(tb, 1)

  plsc.subcore_barrier()
  pltpu.sync_copy(acc_sh.at[pl.ds(s * RPT, RPT)],
                  acc_h.at[c, pl.ds(s * RPT, RPT)])


@jax.jit
def _edge_phase(xl, xr, src3, dst3, att, zeros):
  mesh = plsc.VectorSubcoreMesh(core_axis_name="c", subcore_axis_name="s",
                                num_cores=NC, num_subcores=NS)
  kern = pl.kernel(
      _edge_body,
      out_type=jax.ShapeDtypeStruct((NC, N, DP), _f32),
      mesh=mesh,
      scratch_types=(
          [pltpu.VMEM((H,), _f32)]
          + [pltpu.VMEM((B,), jnp.int32)] * 4
          + [pltpu.VMEM((2, BH), jnp.int32)] * 2
          + [pltpu.VMEM((B, DP), _f32)] * 2
          + [pltpu.VMEM((BH, H), _f32)] * 2
          + [pltpu.VMEM_SHARED((N, DP), _f32)]
          + [pltpu.SemaphoreType.DMA] * 8
      ),
      compiler_params=pltpu.CompilerParams(use_tc_tiling_on_sc=False,
                                           needs_layout_passes=False),
  )
  return kern(xl, xr, src3, dst3, att, zeros)


def _pad_cols(xl):
  marker = (lax.broadcasted_iota(jnp.int32, (N, DP - H), 1) == 0
            ).astype(_f32)
  return jnp.concatenate([xl, marker], axis=1)


def _proj_body(x_ref, wl_ref, wr_ref, xl_ref, xr_ref):
  x = x_ref[...]
  xl_ref[...] = _pad_cols(jnp.dot(x, wl_ref[...], preferred_element_type=_f32))
  xr_ref[...] = jnp.dot(x, wr_ref[...], preferred_element_type=_f32)


def _proj(x, wl, wr):
  return pl.pallas_call(
      _proj_body,
      out_shape=(jax.ShapeDtypeStruct((N, DP), _f32),
                 jax.ShapeDtypeStruct((N, H), _f32)),
  )(x, wl, wr)


def _combine_proj_body(acc_ref, b_ref, wl_ref, wr_ref, xl_ref, xr_ref):
  num = acc_ref[0, :, :H] + acc_ref[1, :, :H]
  den = acc_ref[0, :, H:H + 1] + acc_ref[1, :, H:H + 1]
  h = jnp.where(den > 0.0, num / den, 0.0) + b_ref[...]
  h = jnp.maximum(h, 0.0)
  xl_ref[...] = _pad_cols(jnp.dot(h, wl_ref[...], preferred_element_type=_f32))
  xr_ref[...] = jnp.dot(h, wr_ref[...], preferred_element_type=_f32)


def _combine_proj(acc, b, wl, wr):
  return pl.pallas_call(
      _combine_proj_body,
      out_shape=(jax.ShapeDtypeStruct((N, DP), _f32),
                 jax.ShapeDtypeStruct((N, H), _f32)),
  )(acc, b, wl, wr)


def _head_body(acc_ref, b_ref, batch_ref, wm1_ref, bm1_ref, wm2_ref,
               bm2_ref, wm3_ref, bm3_ref, out_ref):
  num = acc_ref[0, :, :H] + acc_ref[1, :, :H]
  den = acc_ref[0, :, H:H + 1] + acc_ref[1, :, H:H + 1]
  h = jnp.where(den > 0.0, num / den, 0.0) + b_ref[...]
  bi = batch_ref[...]
  seg = lax.broadcasted_iota(jnp.int32, (G, N), 0)
  mask = (bi == seg).astype(_f32)
  sums = jnp.dot(mask, h, preferred_element_type=_f32)
  cnt = jnp.sum(mask, axis=1, keepdims=True)
  pooled = sums / jnp.maximum(cnt, 1.0)
  z = jax.nn.sigmoid(jnp.dot(pooled, wm1_ref[...],
                             preferred_element_type=_f32) + bm1_ref[...])
  z = jax.nn.sigmoid(jnp.dot(z, wm2_ref[...],
                             preferred_element_type=_f32) + bm2_ref[...])
  out_ref[...] = jnp.dot(z, wm3_ref[...],
                         preferred_element_type=_f32) + bm3_ref[...]


def _head(acc, b3, batch, wm1, bm1, wm2, bm2, wm3, bm3):
  return pl.pallas_call(
      _head_body,
      out_shape=jax.ShapeDtypeStruct((G, 2), _f32),
  )(acc, b3, batch, wm1, bm1, wm2, bm2, wm3, bm3)


def kernel(x, edge_index, batch, Wl1, Wr1, att1, b1, Wl2, Wr2, att2, b2,
           Wl3, Wr3, att3, b3, Wm1, bm1, Wm2, bm2, Wm3, bm3):
  src3 = edge_index[0].reshape(NW, NCHUNK, B)
  dst3 = edge_index[1].reshape(NW, NCHUNK, B)
  zeros = jnp.zeros((N, DP), _f32)

  xl, xr = _proj(x, Wl1, Wr1)
  acc = _edge_phase(xl, xr, src3, dst3, att1, zeros)
  xl, xr = _combine_proj(acc, b1.reshape(1, H), Wl2, Wr2)
  acc = _edge_phase(xl, xr, src3, dst3, att2, zeros)
  xl, xr = _combine_proj(acc, b2.reshape(1, H), Wl3, Wr3)
  acc = _edge_phase(xl, xr, src3, dst3, att3, zeros)
  return _head(acc, b3.reshape(1, H), batch.reshape(1, N),
               Wm1, bm1.reshape(1, -1), Wm2, bm2.reshape(1, -1),
               Wm3, bm3.reshape(1, 2))
